# baseline (speedup 1.0000x reference)
import jax
import jax.numpy as jnp
from jax import lax
from jax.experimental import pallas as pl
from jax.experimental.pallas import tpu as pltpu
from jax.experimental.pallas import tpu_sc as plsc

_NC, _NS = 2, 16
_CHUNK = 200
_PAD_ROWS = 8000
_LANES = 16


def _pad_table(table):
    v, d = table.shape

    def body(t_ref, o_ref):
        o_ref[:, :d] = t_ref[...]

    return pl.pallas_call(
        body,
        grid=(v // _PAD_ROWS,),
        in_specs=[pl.BlockSpec((_PAD_ROWS, d), lambda i: (i, 0))],
        out_specs=pl.BlockSpec((_PAD_ROWS, 128), lambda i: (i, 0)),
        out_shape=jax.ShapeDtypeStruct((v, 128), table.dtype),
    )(table)


def kernel(inputs, table):
    b, s = inputs.shape
    n = b * s
    v, d = table.shape
    nw = _NC * _NS
    per_w = n // nw
    n_chunks = per_w // _CHUNK
    n_pairs = n_chunks // 2
    idx = inputs.reshape(n).astype(jnp.int32)
    table_pad = _pad_table(table)
    mesh = plsc.VectorSubcoreMesh(core_axis_name="c", subcore_axis_name="s")

    @pl.kernel(
        out_type=jax.ShapeDtypeStruct((n, d), table.dtype),
        mesh=mesh,
        scratch_types=[
            pltpu.VMEM((_CHUNK,), jnp.int32),
            pltpu.VMEM((_CHUNK,), jnp.int32),
            pltpu.VMEM((_CHUNK, 128), jnp.float32),
            pltpu.VMEM((_CHUNK, 128), jnp.float32),
            pltpu.VMEM((_CHUNK, d), jnp.float32),
            pltpu.VMEM((_CHUNK, d), jnp.float32),
            pltpu.SemaphoreType.DMA,
            pltpu.SemaphoreType.DMA,
            pltpu.SemaphoreType.DMA,
            pltpu.SemaphoreType.DMA,
        ],
    )
    def gather_kernel(table_hbm, idx_hbm, out_hbm,
                      i0, i1, r0, r1, o0, o1, g0, g1, w0, w1):
        wid = lax.axis_index("s") * _NC + lax.axis_index("c")
        base = wid * per_w
        ibufs, rbufs, obufs = (i0, i1), (r0, r1), (o0, o1)
        gsems, wsems = (g0, g1), (w0, w1)

        def compact(p):
            r_ref, o_ref = rbufs[p], obufs[p]

            @plsc.parallel_loop(0, _CHUNK)
            def _(j):
                for q in range(d // _LANES):
                    o_ref[j, pl.ds(q * _LANES, _LANES)] = (
                        r_ref[j, pl.ds(q * _LANES, _LANES)])

        def wait_gather(p):
            pltpu.make_async_copy(table_hbm.at[ibufs[p]], rbufs[p],
                                  gsems[p]).wait()

        def wait_write(p, coff):
            pltpu.make_async_copy(obufs[p], out_hbm.at[pl.ds(coff, _CHUNK)],
                                  wsems[p]).wait()

        for p in range(2):
            pltpu.sync_copy(idx_hbm.at[pl.ds(base + p * _CHUNK, _CHUNK)],
                            ibufs[p])
            pltpu.async_copy(table_hbm.at[ibufs[p]], rbufs[p], gsems[p])

        for p in range(2):
            coff = base + p * _CHUNK
            wait_gather(p)
            compact(p)
            pltpu.async_copy(obufs[p], out_hbm.at[pl.ds(coff, _CHUNK)],
                             wsems[p])
            pltpu.sync_copy(idx_hbm.at[pl.ds(coff + 2 * _CHUNK, _CHUNK)],
                            ibufs[p])
            pltpu.async_copy(table_hbm.at[ibufs[p]], rbufs[p], gsems[p])

        @pl.loop(1, n_pairs - 1)
        def _(pair):
            off = base + 2 * pair * _CHUNK
            for p in range(2):
                coff = off + p * _CHUNK
                wait_gather(p)
                wait_write(p, coff - 2 * _CHUNK)
                compact(p)
                pltpu.async_copy(obufs[p], out_hbm.at[pl.ds(coff, _CHUNK)],
                                 wsems[p])
                pltpu.sync_copy(idx_hbm.at[pl.ds(coff + 2 * _CHUNK, _CHUNK)],
                                ibufs[p])
                pltpu.async_copy(table_hbm.at[ibufs[p]], rbufs[p], gsems[p])

        last = base + (n_chunks - 2) * _CHUNK
        for p in range(2):
            coff = last + p * _CHUNK
            wait_gather(p)
            wait_write(p, coff - 2 * _CHUNK)
            compact(p)
            pltpu.async_copy(obufs[p], out_hbm.at[pl.ds(coff, _CHUNK)],
                             wsems[p])
        for p in range(2):
            wait_write(p, last + p * _CHUNK)

    out = gather_kernel(table_pad, idx)
    return out.reshape(b, s, d)


# speedup vs baseline: 1.0008x; 1.0008x over previous
import jax
import jax.numpy as jnp
from jax import lax
from jax.experimental import pallas as pl
from jax.experimental.pallas import tpu as pltpu
from jax.experimental.pallas import tpu_sc as plsc
from jax.experimental.layout import Layout, with_layout_constraint

_NC, _NS = 2, 16
_CHUNK = 400
_PAD_ROWS = 8000


def _pad_table(table):
    v, d = table.shape

    def body(t_ref, o_ref):
        o_ref[:, :d] = t_ref[...]

    return pl.pallas_call(
        body,
        grid=(v // _PAD_ROWS,),
        in_specs=[pl.BlockSpec((_PAD_ROWS, d), lambda i: (i, 0))],
        out_specs=pl.BlockSpec((_PAD_ROWS, 128), lambda i: (i, 0)),
        out_shape=jax.ShapeDtypeStruct((v, 128), table.dtype),
    )(table)


def kernel(inputs, table):
    b, s = inputs.shape
    n = b * s
    v, d = table.shape
    nw = _NC * _NS
    per_w = n // nw
    n_chunks = per_w // _CHUNK
    n_pairs = n_chunks // 2
    idx = inputs.reshape(n).astype(jnp.int32)
    table_pad = with_layout_constraint(
        _pad_table(table),
        Layout(major_to_minor=(0, 1), tiling=((8, 128),)))
    mesh = plsc.VectorSubcoreMesh(core_axis_name="c", subcore_axis_name="s")

    @pl.kernel(
        out_type=jax.ShapeDtypeStruct((n, 128), table.dtype),
        mesh=mesh,
        scratch_types=[
            pltpu.VMEM((_CHUNK,), jnp.int32),
            pltpu.VMEM((_CHUNK,), jnp.int32),
            pltpu.VMEM((_CHUNK, 128), jnp.float32),
            pltpu.VMEM((_CHUNK, 128), jnp.float32),
            pltpu.SemaphoreType.DMA,
            pltpu.SemaphoreType.DMA,
            pltpu.SemaphoreType.DMA,
            pltpu.SemaphoreType.DMA,
        ],
    )
    def gather_kernel(table_hbm, idx_hbm, out_hbm,
                      i0, i1, r0, r1, g0, g1, w0, w1):
        wid = lax.axis_index("s") * _NC + lax.axis_index("c")
        base = wid * per_w
        ibufs, rbufs, gsems, wsems = (i0, i1), (r0, r1), (g0, g1), (w0, w1)

        for p in range(2):
            pltpu.sync_copy(idx_hbm.at[pl.ds(base + p * _CHUNK, _CHUNK)],
                            ibufs[p])
            pltpu.async_copy(table_hbm.at[ibufs[p]], rbufs[p], gsems[p])

        @pl.loop(0, n_pairs - 1)
        def _(pair):
            off = base + 2 * pair * _CHUNK
            for p in range(2):
                coff = off + p * _CHUNK
                pltpu.make_async_copy(table_hbm.at[ibufs[p]], rbufs[p],
                                      gsems[p]).wait()
                pltpu.async_copy(rbufs[p], out_hbm.at[pl.ds(coff, _CHUNK)],
                                 wsems[p])
                pltpu.sync_copy(
                    idx_hbm.at[pl.ds(coff + 2 * _CHUNK, _CHUNK)], ibufs[p])
                pltpu.make_async_copy(rbufs[p],
                                      out_hbm.at[pl.ds(coff, _CHUNK)],
                                      wsems[p]).wait()
                pltpu.async_copy(table_hbm.at[ibufs[p]], rbufs[p], gsems[p])

        last = base + (n_chunks - 2) * _CHUNK
        for p in range(2):
            coff = last + p * _CHUNK
            pltpu.make_async_copy(table_hbm.at[ibufs[p]], rbufs[p],
                                  gsems[p]).wait()
            pltpu.async_copy(rbufs[p], out_hbm.at[pl.ds(coff, _CHUNK)],
                             wsems[p])
        for p in range(2):
            coff = last + p * _CHUNK
            pltpu.make_async_copy(rbufs[p], out_hbm.at[pl.ds(coff, _CHUNK)],
                                  wsems[p]).wait()

    out = gather_kernel(table_pad, idx)
    return out[:, :d].reshape(b, s, d)
